# baseline (speedup 1.0000x reference)
import functools

import jax
import jax.numpy as jnp
from jax import lax
from jax.experimental import pallas as pl
from jax.experimental.pallas import tpu as pltpu
from jax.experimental.pallas import tpu_sc as plsc

V = 1_000_000
B = 128
VL = 1024
CH = 65536
NBLK = V // CH
TAIL = V - NBLK * CH
FAN = 8

_mesh = plsc.VectorSubcoreMesh(
    core_axis_name="c", subcore_axis_name="s", num_cores=1, num_subcores=1
)


@functools.partial(
    pl.kernel,
    out_type=jax.ShapeDtypeStruct((B,), jnp.float32),
    mesh=_mesh,
    scratch_types=[
        pltpu.VMEM((B,), jnp.int32),
        pltpu.VMEM((B,), jnp.float32),
        pltpu.SemaphoreType.DMA,
    ],
)
def _sc_gather(logits_hbm, value_hbm, out_g, idx_v, g_v, sem):
    sid = lax.axis_index("s")
    cid = lax.axis_index("c")

    @pl.when((sid == 0) & (cid == 0))
    def _():
        pltpu.sync_copy(value_hbm, idx_v)
        pltpu.async_copy(logits_hbm.at[idx_v], g_v, sem).wait()
        pltpu.sync_copy(g_v, out_g)


NG = 122


def _norm_body(x_hbm, o_ref, stash_ref, sem):
    pltpu.make_async_copy(x_hbm, stash_ref, sem).start()
    pltpu.make_async_copy(x_hbm, stash_ref, sem).wait()

    def max_body(j, accs):
        base = j * (FAN * VL)
        return tuple(
            jnp.maximum(accs[k], stash_ref[pl.ds(base + k * VL, VL)])
            for k in range(FAN)
        )

    minf = jnp.full((VL,), -jnp.inf, jnp.float32)
    maccs = lax.fori_loop(0, NG, max_body, (minf,) * FAN)
    tail = stash_ref[pl.ds(NG * FAN * VL, V - NG * FAN * VL)]
    gmax = jnp.maximum(jnp.max(functools.reduce(jnp.maximum, maccs)),
                       jnp.max(tail))

    def sum_body(j, accs):
        base = j * (FAN * VL)
        return tuple(
            accs[k] + jnp.exp(stash_ref[pl.ds(base + k * VL, VL)] - gmax)
            for k in range(FAN)
        )

    zero = jnp.zeros((VL,), jnp.float32)
    saccs = lax.fori_loop(0, NG, sum_body, (zero,) * FAN)
    total = jnp.sum(functools.reduce(jnp.add, saccs))
    total = total + jnp.sum(jnp.exp(tail - gmax))
    o_ref[...] = jnp.full((B,), gmax + jnp.log(total), jnp.float32)


def _tc_norm(x1d):
    return pl.pallas_call(
        _norm_body,
        in_specs=[pl.BlockSpec(memory_space=pl.ANY)],
        out_shape=jax.ShapeDtypeStruct((B,), jnp.float32),
        scratch_shapes=[
            pltpu.VMEM((V,), jnp.float32),
            pltpu.SemaphoreType.DMA,
        ],
    )(x1d)


def kernel(logits, value):
    g = _sc_gather(logits, value)
    norm = _tc_norm(logits)
    return g - norm


# speedup vs baseline: 1.0397x; 1.0397x over previous
import functools

import jax
import jax.numpy as jnp
from jax import lax
from jax.experimental import pallas as pl
from jax.experimental.pallas import tpu as pltpu
from jax.experimental.pallas import tpu_sc as plsc

V = 1_000_000
B = 128
VL = 1024
CH = 65536
NBLK = V // CH
TAIL = V - NBLK * CH
FAN = 8

_mesh = plsc.VectorSubcoreMesh(
    core_axis_name="c", subcore_axis_name="s", num_cores=1, num_subcores=1
)


@functools.partial(
    pl.kernel,
    out_type=jax.ShapeDtypeStruct((B,), jnp.float32),
    mesh=_mesh,
    scratch_types=[
        pltpu.VMEM((B,), jnp.int32),
        pltpu.VMEM((B,), jnp.float32),
        pltpu.SemaphoreType.DMA,
    ],
)
def _sc_gather(logits_hbm, value_hbm, out_g, idx_v, g_v, sem):
    sid = lax.axis_index("s")
    cid = lax.axis_index("c")

    @pl.when((sid == 0) & (cid == 0))
    def _():
        pltpu.sync_copy(value_hbm, idx_v)
        pltpu.async_copy(logits_hbm.at[idx_v], g_v, sem).wait()
        pltpu.sync_copy(g_v, out_g)


CNK = 122880
NCHK = 8
LAST_EXTRA = V - NCHK * CNK
NGC = CNK // (FAN * VL)


def _norm_body(x_hbm, o_ref, stash_ref, sems):
    copies = []
    for c in range(NCHK):
        ln = CNK + (LAST_EXTRA if c == NCHK - 1 else 0)
        cp = pltpu.make_async_copy(
            x_hbm.at[pl.ds(c * CNK, ln)],
            stash_ref.at[pl.ds(c * CNK, ln)],
            sems.at[c],
        )
        cp.start()
        copies.append(cp)

    minf = jnp.full((VL,), -jnp.inf, jnp.float32)
    zero = jnp.zeros((VL,), jnp.float32)
    ms, ss = [], []
    for c in range(NCHK):
        copies[c].wait()
        base0 = c * CNK

        def max_body(j, accs, base0=base0):
            base = base0 + j * (FAN * VL)
            return tuple(
                jnp.maximum(accs[k], stash_ref[pl.ds(base + k * VL, VL)])
                for k in range(FAN)
            )

        maccs = lax.fori_loop(0, NGC, max_body, (minf,) * FAN)
        m_c = functools.reduce(jnp.maximum, maccs)

        def sum_body(j, accs, base0=base0, m_c=m_c):
            base = base0 + j * (FAN * VL)
            return tuple(
                accs[k] + jnp.exp(stash_ref[pl.ds(base + k * VL, VL)] - m_c)
                for k in range(FAN)
            )

        saccs = lax.fori_loop(0, NGC, sum_body, (zero,) * FAN)
        ms.append(m_c)
        ss.append(functools.reduce(jnp.add, saccs))

    tail = stash_ref[pl.ds(NCHK * CNK, LAST_EXTRA)]
    gmax = jnp.maximum(jnp.max(functools.reduce(jnp.maximum, ms)),
                       jnp.max(tail))
    acc = zero
    for m_c, s_c in zip(ms, ss):
        acc = acc + s_c * jnp.exp(m_c - gmax)
    total = jnp.sum(acc) + jnp.sum(jnp.exp(tail - gmax))
    o_ref[...] = jnp.full((B,), gmax + jnp.log(total), jnp.float32)


def _tc_norm(x1d):
    return pl.pallas_call(
        _norm_body,
        in_specs=[pl.BlockSpec(memory_space=pl.ANY)],
        out_shape=jax.ShapeDtypeStruct((B,), jnp.float32),
        scratch_shapes=[
            pltpu.VMEM((V,), jnp.float32),
            pltpu.SemaphoreType.DMA((NCHK,)),
        ],
    )(x1d)


def kernel(logits, value):
    g = _sc_gather(logits, value)
    norm = _tc_norm(logits)
    return g - norm
